# baseline (speedup 1.0000x reference)
import functools

import jax
import jax.numpy as jnp
from jax import lax
from jax.experimental import pallas as pl
from jax.experimental.pallas import tpu as pltpu
from jax.experimental.pallas import tpu_sc as plsc

B = 16384
S = 5
D = 64
H = 128

NC = 2
NS = 16
NW = NC * NS
BPW = B // NW

CB = 16
ROWS = CB * S
NCH = BPW // CB


def _sc_pool_body(x_hbm, emb_hbm, out_hbm, idx_v, rows_v, pooled_v, sem):
    wid = lax.axis_index("s") * NC + lax.axis_index("c")
    base = wid * BPW
    pltpu.sync_copy(x_hbm.at[pl.ds(base * S, BPW * S)], idx_v)

    def chunk(g, _):
        pltpu.async_copy(
            emb_hbm.at[idx_v.at[pl.ds(g * ROWS, ROWS)]], rows_v, sem
        ).wait()

        def pool_one(i, _):
            r = i * S
            for q in range(D // 16):
                col = pl.ds(q * 16, 16)
                acc = rows_v[r, col]
                for s in range(1, S):
                    acc = acc + rows_v[r + s, col]
                pooled_v[g * CB + i, col] = acc * (1.0 / S)
            return 0

        lax.fori_loop(0, CB, pool_one, 0)
        return 0

    lax.fori_loop(0, NCH, chunk, 0)
    pltpu.sync_copy(pooled_v, out_hbm.at[pl.ds(base, BPW)])


@functools.partial(jax.jit, static_argnames=())
def _sc_pool(x_flat, embedding):
    mesh = plsc.VectorSubcoreMesh(core_axis_name="c", subcore_axis_name="s")
    return pl.kernel(
        _sc_pool_body,
        out_type=jax.ShapeDtypeStruct((B, D), jnp.float32),
        mesh=mesh,
        scratch_types=[
            pltpu.VMEM((BPW * S,), jnp.int32),
            pltpu.VMEM((ROWS, D), jnp.float32),
            pltpu.VMEM((BPW, D), jnp.float32),
            pltpu.SemaphoreType.DMA,
        ],
        compiler_params=pltpu.CompilerParams(use_tc_tiling_on_sc=False),
    )(x_flat, embedding)


def _mlp_body(p_ref, w1_ref, b1_ref, w2_ref, b2_ref, o_ref):
    h = jnp.dot(p_ref[...], w1_ref[...], preferred_element_type=jnp.float32)
    h = jnp.maximum(h + b1_ref[...], 0.0)
    o = jnp.dot(h, w2_ref[...], preferred_element_type=jnp.float32)
    o = o + b2_ref[...]
    o_ref[...] = jnp.clip(o, 0.0, 100.0)


MB = 2048


def _mlp(pooled, W1, b1, W2, b2):
    return pl.pallas_call(
        _mlp_body,
        grid=(B // MB,),
        in_specs=[
            pl.BlockSpec((MB, D), lambda i: (i, 0)),
            pl.BlockSpec((D, H), lambda i: (0, 0)),
            pl.BlockSpec((1, H), lambda i: (0, 0)),
            pl.BlockSpec((H, 1), lambda i: (0, 0)),
            pl.BlockSpec((1, 1), lambda i: (0, 0)),
        ],
        out_specs=pl.BlockSpec((MB, 1), lambda i: (i, 0)),
        out_shape=jax.ShapeDtypeStruct((B, 1), jnp.float32),
    )(pooled, W1, b1.reshape(1, H), W2, b2.reshape(1, 1))


def kernel(x, embedding, W1, b1, W2, b2):
    pooled = _sc_pool(x.reshape(-1), embedding)
    out = _mlp(pooled, W1, b1, W2, b2)
    return out.reshape(B)
